# initial kernel scaffold (unmeasured)
import jax
import jax.numpy as jnp
from jax import lax
from jax.experimental import pallas as pl
from jax.experimental.pallas import tpu as pltpu

B, S, H, Dh, Dr = 2, 512, 16, 128, 32
D = 2048
DC = 128
BS = B * S
SCALE = (Dh + Dr) ** -0.5
BF16 = jnp.bfloat16


def _mm(a, b):
    return lax.dot_general(a, b, (((1,), (0,)), ((), ())),
                           preferred_element_type=jnp.float32)


def _mmT(a, b):
    return lax.dot_general(a, b, (((1,), (1,)), ((), ())),
                           preferred_element_type=jnp.float32)


def kernel(x, Wdkv, Wuk, Wuv, Wq, Wqr, Wkr, Wo):
    def body(x_ref, wdkv_ref, wuk_ref, wuv_ref, wq_ref, wqr_ref, wkr_ref,
             wo_ref, out_ref,
             c_self, c_other, w_self, w_other,
             q_ref, qr_ref, kr_ref, k_ref, v_ref, attn_ref,
             send_sems, recv_sems):
        my_x = lax.axis_index("x")
        my_y = lax.axis_index("y")
        my_z = lax.axis_index("z")
        partner = (1 - my_x, my_y, my_z)

        barrier = pltpu.get_barrier_semaphore()
        pl.semaphore_signal(barrier, inc=1, device_id=partner,
                            device_id_type=pl.DeviceIdType.MESH)
        pl.semaphore_wait(barrier, 1)

        w_self[0] = wuk_ref[:].astype(BF16)
        w_self[1] = wuv_ref[:].astype(BF16)
        w_rdma = pltpu.make_async_remote_copy(
            src_ref=w_self, dst_ref=w_other,
            send_sem=send_sems.at[0], recv_sem=recv_sems.at[0],
            device_id=partner, device_id_type=pl.DeviceIdType.MESH)
        w_rdma.start()

        wdkv = wdkv_ref[:].astype(BF16)
        for b in range(B):
            xb = x_ref[b].astype(BF16)
            c_self[pl.ds(b * S, S), :] = _mm(xb, wdkv).astype(BF16)
        c_rdma = pltpu.make_async_remote_copy(
            src_ref=c_self, dst_ref=c_other,
            send_sem=send_sems.at[1], recv_sem=recv_sems.at[1],
            device_id=partner, device_id_type=pl.DeviceIdType.MESH)
        c_rdma.start()

        wq = wq_ref[:].astype(BF16)
        wqr = wqr_ref[:].astype(BF16)
        wkr = wkr_ref[:].astype(BF16)
        for b in range(B):
            xb = x_ref[b].astype(BF16)
            q_ref[pl.ds(b * S, S), :] = _mm(xb, wq).astype(BF16)
            qr_ref[pl.ds(b * S, S), :] = _mm(xb, wqr).astype(BF16)
            kr_ref[pl.ds(b * S, S), :] = _mm(xb, wkr).astype(BF16)

        w_rdma.wait()
        c_rdma.wait()

        for b in range(B):
            cs = c_self[pl.ds(b * S, S), :]
            co = c_other[pl.ds(b * S, S), :]
            k_ref[pl.ds(b * S, S), :] = (
                _mm(cs, w_self[0]) + _mm(co, w_other[0])).astype(BF16)
            v_ref[pl.ds(b * S, S), :] = (
                _mm(cs, w_self[1]) + _mm(co, w_other[1])).astype(BF16)

        def attn_step(i, carry):
            b = i // H
            h = i - b * H
            ro = b * S
            co = h * Dh
            q = q_ref[pl.ds(ro, S), pl.ds(co, Dh)]
            k = k_ref[pl.ds(ro, S), pl.ds(co, Dh)]
            qr = qr_ref[pl.ds(ro, S), pl.ds(h * Dr, Dr)]
            kr = kr_ref[pl.ds(ro, S), :]
            s = (_mmT(q, k) + _mmT(qr, kr)) * SCALE
            m = jnp.max(s, axis=1, keepdims=True)
            p = jnp.exp(s - m)
            p = p / jnp.sum(p, axis=1, keepdims=True)
            o = _mm(p.astype(BF16), v_ref[pl.ds(ro, S), pl.ds(co, Dh)])
            attn_ref[pl.ds(ro, S), pl.ds(co, Dh)] = o.astype(BF16)
            return carry

        lax.fori_loop(0, B * H, attn_step, 0)

        wo = wo_ref[:].astype(BF16)
        for b in range(B):
            out_ref[b] = _mm(attn_ref[pl.ds(b * S, S), :], wo)

    vmem = pl.BlockSpec(memory_space=pltpu.VMEM)
    return pl.pallas_call(
        body,
        out_shape=jax.ShapeDtypeStruct((B, S, D), jnp.float32),
        in_specs=[vmem] * 8,
        out_specs=vmem,
        scratch_shapes=[
            pltpu.VMEM((BS, DC), BF16),
            pltpu.VMEM((BS, DC), BF16),
            pltpu.VMEM((2, DC, D), BF16),
            pltpu.VMEM((2, DC, D), BF16),
            pltpu.VMEM((BS, D), BF16),
            pltpu.VMEM((BS, H * Dr), BF16),
            pltpu.VMEM((BS, Dr), BF16),
            pltpu.VMEM((BS, D), BF16),
            pltpu.VMEM((BS, D), BF16),
            pltpu.VMEM((BS, D), BF16),
            pltpu.SemaphoreType.DMA((2,)),
            pltpu.SemaphoreType.DMA((2,)),
        ],
        compiler_params=pltpu.CompilerParams(collective_id=0),
    )(x, Wdkv, Wuk, Wuv, Wq, Wqr, Wkr, Wo)


# baseline (device time: 117817 ns/iter reference)
import jax
import jax.numpy as jnp
from jax import lax
from jax.experimental import pallas as pl
from jax.experimental.pallas import tpu as pltpu

B, S, H, Dh, Dr = 2, 512, 16, 128, 32
D = 2048
DC = 128
BS = B * S
SCALE = (Dh + Dr) ** -0.5
BF16 = jnp.bfloat16


def _mm(a, b):
    return lax.dot_general(a, b, (((1,), (0,)), ((), ())),
                           preferred_element_type=jnp.float32)


def _mmT(a, b):
    return lax.dot_general(a, b, (((1,), (1,)), ((), ())),
                           preferred_element_type=jnp.float32)


def kernel(x, Wdkv, Wuk, Wuv, Wq, Wqr, Wkr, Wo):
    def body(x_ref, wdkv_ref, wuk_ref, wuv_ref, wq_ref, wqr_ref, wkr_ref,
             wo_ref, out_ref,
             c_self, c_other, w_other,
             q_ref, qr_ref, kr_ref, k_ref, v_ref, attn_ref,
             send_sems, recv_sems):
        my_x = lax.axis_index("x")
        my_y = lax.axis_index("y")
        my_z = lax.axis_index("z")
        partner = (1 - my_x, my_y, my_z)

        barrier = pltpu.get_barrier_semaphore()
        pl.semaphore_signal(barrier, inc=1, device_id=partner,
                            device_id_type=pl.DeviceIdType.MESH)
        pl.semaphore_wait(barrier, 1)

        wuk_rdma = pltpu.make_async_remote_copy(
            src_ref=wuk_ref, dst_ref=w_other.at[0],
            send_sem=send_sems.at[0], recv_sem=recv_sems.at[0],
            device_id=partner, device_id_type=pl.DeviceIdType.MESH)
        wuk_rdma.start()
        wuv_rdma = pltpu.make_async_remote_copy(
            src_ref=wuv_ref, dst_ref=w_other.at[1],
            send_sem=send_sems.at[1], recv_sem=recv_sems.at[1],
            device_id=partner, device_id_type=pl.DeviceIdType.MESH)
        wuv_rdma.start()

        for b in range(B):
            c_self[pl.ds(b * S, S), :] = _mm(x_ref[b], wdkv_ref[:]).astype(BF16)
        c_rdma = pltpu.make_async_remote_copy(
            src_ref=c_self, dst_ref=c_other,
            send_sem=send_sems.at[2], recv_sem=recv_sems.at[2],
            device_id=partner, device_id_type=pl.DeviceIdType.MESH)
        c_rdma.start()

        NT = 512
        for b in range(B):
            xb = x_ref[b]
            for n0 in range(0, D, NT):
                q_ref[pl.ds(b * S, S), pl.ds(n0, NT)] = _mm(
                    xb, wq_ref[:, pl.ds(n0, NT)]).astype(BF16)
            qrb = _mm(xb, wqr_ref[:]).astype(BF16)
            for h in range(H):
                qr_ref[h, pl.ds(b * S, S), :] = qrb[:, h * Dr:(h + 1) * Dr]
            kr_ref[pl.ds(b * S, S), :] = _mm(xb, wkr_ref[:]).astype(BF16)

        wuk_rdma.wait()
        wuv_rdma.wait()
        c_rdma.wait()

        for b in range(B):
            cs = c_self[pl.ds(b * S, S), :]
            co = c_other[pl.ds(b * S, S), :]
            for n0 in range(0, D, NT):
                cols = pl.ds(n0, NT)
                k_ref[pl.ds(b * S, S), cols] = (
                    _mm(cs, wuk_ref[:, cols])
                    + _mm(co, w_other[0, :, cols])).astype(BF16)
                v_ref[pl.ds(b * S, S), cols] = (
                    _mm(cs, wuv_ref[:, cols])
                    + _mm(co, w_other[1, :, cols])).astype(BF16)

        def attn_step(i, carry):
            b = i // H
            h = i - b * H
            ro = b * S
            co = h * Dh
            q = q_ref[pl.ds(ro, S), pl.ds(co, Dh)]
            k = k_ref[pl.ds(ro, S), pl.ds(co, Dh)]
            qr = qr_ref[h, pl.ds(ro, S), :]
            kr = kr_ref[pl.ds(ro, S), :]
            s = (_mmT(q, k) + _mmT(qr, kr)) * SCALE
            m = jnp.max(s, axis=1, keepdims=True)
            p = jnp.exp(s - m)
            p = p / jnp.sum(p, axis=1, keepdims=True)
            o = _mm(p.astype(BF16), v_ref[pl.ds(ro, S), pl.ds(co, Dh)])
            attn_ref[pl.ds(ro, S), pl.ds(co, Dh)] = o.astype(BF16)
            return carry

        lax.fori_loop(0, B * H, attn_step, 0)

        for b in range(B):
            ab = attn_ref[pl.ds(b * S, S), :]
            for n0 in range(0, D, NT):
                out_ref[b, :, pl.ds(n0, NT)] = _mm(
                    ab, wo_ref[:, pl.ds(n0, NT)])

    vmem = pl.BlockSpec(memory_space=pltpu.VMEM)
    args = [a.astype(BF16) for a in (x, Wdkv, Wuk, Wuv, Wq, Wqr, Wkr, Wo)]
    return pl.pallas_call(
        body,
        out_shape=jax.ShapeDtypeStruct((B, S, D), jnp.float32),
        in_specs=[vmem] * 8,
        out_specs=vmem,
        scratch_shapes=[
            pltpu.VMEM((BS, DC), BF16),
            pltpu.VMEM((BS, DC), BF16),
            pltpu.VMEM((2, DC, D), BF16),
            pltpu.VMEM((BS, D), BF16),
            pltpu.VMEM((H, BS, Dr), BF16),
            pltpu.VMEM((BS, Dr), BF16),
            pltpu.VMEM((BS, D), BF16),
            pltpu.VMEM((BS, D), BF16),
            pltpu.VMEM((BS, D), BF16),
            pltpu.SemaphoreType.DMA((3,)),
            pltpu.SemaphoreType.DMA((3,)),
        ],
        compiler_params=pltpu.CompilerParams(
            collective_id=0,
            vmem_limit_bytes=60 * 1024 * 1024,
        ),
    )(*args)


# device time: 97066 ns/iter; 1.2138x vs baseline; 1.2138x over previous
import jax
import jax.numpy as jnp
from jax import lax
from jax.experimental import pallas as pl
from jax.experimental.pallas import tpu as pltpu

B, S, H, Dh, Dr = 2, 512, 16, 128, 32
D = 2048
DC = 128
BS = B * S
SCALE = (Dh + Dr) ** -0.5
BF16 = jnp.bfloat16
NT = 512
NSTRIP = D // NT


def _mm(a, b):
    return lax.dot_general(a, b, (((1,), (0,)), ((), ())),
                           preferred_element_type=jnp.float32)


def _mmT(a, b):
    return lax.dot_general(a, b, (((1,), (1,)), ((), ())),
                           preferred_element_type=jnp.float32)


def kernel(x, Wdkv, Wuk, Wuv, Wq, Wqr, Wkr, Wo):
    def body(x_ref, wdkv_ref, wuk_ref, wuv_ref, wq_hbm, wqr_ref, wkr_ref,
             wo_hbm, out_ref,
             c_self, c_other, w_other,
             q_ref, qr_ref, kr_ref, k_ref, v_ref, attn_ref, strip_buf,
             send_sems, recv_sems, strip_sems):
        my_x = lax.axis_index("x")
        my_y = lax.axis_index("y")
        my_z = lax.axis_index("z")
        partner = (1 - my_x, my_y, my_z)

        def strip_cp(hbm_ref, n):
            return pltpu.make_async_copy(
                hbm_ref.at[:, pl.ds(n * NT, NT)],
                strip_buf.at[n % 2], strip_sems.at[n % 2])

        strip_cp(wq_hbm, 0).start()

        barrier = pltpu.get_barrier_semaphore()
        pl.semaphore_signal(barrier, inc=1, device_id=partner,
                            device_id_type=pl.DeviceIdType.MESH)
        pl.semaphore_wait(barrier, 1)

        wuk_rdma = pltpu.make_async_remote_copy(
            src_ref=wuk_ref, dst_ref=w_other.at[0],
            send_sem=send_sems.at[0], recv_sem=recv_sems.at[0],
            device_id=partner, device_id_type=pl.DeviceIdType.MESH)
        wuk_rdma.start()
        wuv_rdma = pltpu.make_async_remote_copy(
            src_ref=wuv_ref, dst_ref=w_other.at[1],
            send_sem=send_sems.at[1], recv_sem=recv_sems.at[1],
            device_id=partner, device_id_type=pl.DeviceIdType.MESH)
        wuv_rdma.start()

        for b in range(B):
            c_self[pl.ds(b * S, S), :] = _mm(x_ref[b], wdkv_ref[:]).astype(BF16)
        c_rdma = pltpu.make_async_remote_copy(
            src_ref=c_self, dst_ref=c_other,
            send_sem=send_sems.at[2], recv_sem=recv_sems.at[2],
            device_id=partner, device_id_type=pl.DeviceIdType.MESH)
        c_rdma.start()

        for n in range(NSTRIP):
            if n + 1 < NSTRIP:
                strip_cp(wq_hbm, n + 1).start()
            strip_cp(wq_hbm, n).wait()
            wqs = strip_buf[n % 2].astype(BF16)
            for b in range(B):
                q_ref[pl.ds(b * S, S), pl.ds(n * NT, NT)] = _mm(
                    x_ref[b], wqs).astype(BF16)

        for b in range(B):
            xb = x_ref[b]
            qrb = _mm(xb, wqr_ref[:]).astype(BF16)
            for h in range(H):
                qr_ref[h, pl.ds(b * S, S), :] = qrb[:, h * Dr:(h + 1) * Dr]
            kr_ref[pl.ds(b * S, S), :] = _mm(xb, wkr_ref[:]).astype(BF16)

        wuk_rdma.wait()
        wuv_rdma.wait()
        c_rdma.wait()

        for b in range(B):
            cs = c_self[pl.ds(b * S, S), :]
            co = c_other[pl.ds(b * S, S), :]
            for n0 in range(0, D, NT):
                cols = pl.ds(n0, NT)
                k_ref[pl.ds(b * S, S), cols] = (
                    _mm(cs, wuk_ref[:, cols])
                    + _mm(co, w_other[0, :, cols])).astype(BF16)
                v_ref[pl.ds(b * S, S), cols] = (
                    _mm(cs, wuv_ref[:, cols])
                    + _mm(co, w_other[1, :, cols])).astype(BF16)

        strip_cp(wo_hbm, 0).start()

        def attn_step(i, carry):
            b = i // H
            h = i - b * H
            ro = b * S
            co = h * Dh
            q = q_ref[pl.ds(ro, S), pl.ds(co, Dh)]
            k = k_ref[pl.ds(ro, S), pl.ds(co, Dh)]
            qr = qr_ref[h, pl.ds(ro, S), :]
            kr = kr_ref[pl.ds(ro, S), :]
            s = (_mmT(q, k) + _mmT(qr, kr)) * SCALE
            m = jnp.max(s, axis=1, keepdims=True)
            p = jnp.exp(s - m)
            p = p / jnp.sum(p, axis=1, keepdims=True)
            o = _mm(p.astype(BF16), v_ref[pl.ds(ro, S), pl.ds(co, Dh)])
            attn_ref[pl.ds(ro, S), pl.ds(co, Dh)] = o.astype(BF16)
            return carry

        lax.fori_loop(0, B * H, attn_step, 0)

        for n in range(NSTRIP):
            if n + 1 < NSTRIP:
                strip_cp(wo_hbm, n + 1).start()
            strip_cp(wo_hbm, n).wait()
            wos = strip_buf[n % 2].astype(BF16)
            for b in range(B):
                out_ref[b, :, pl.ds(n * NT, NT)] = _mm(
                    attn_ref[pl.ds(b * S, S), :], wos)

    vmem = pl.BlockSpec(memory_space=pltpu.VMEM)
    hbm = pl.BlockSpec(memory_space=pl.ANY)
    small = [a.astype(BF16) for a in (x, Wdkv, Wuk, Wuv, Wqr, Wkr)]
    return pl.pallas_call(
        body,
        out_shape=jax.ShapeDtypeStruct((B, S, D), jnp.float32),
        in_specs=[vmem, vmem, vmem, vmem, hbm, vmem, vmem, hbm],
        out_specs=vmem,
        scratch_shapes=[
            pltpu.VMEM((BS, DC), BF16),
            pltpu.VMEM((BS, DC), BF16),
            pltpu.VMEM((2, DC, D), BF16),
            pltpu.VMEM((BS, D), BF16),
            pltpu.VMEM((H, BS, Dr), BF16),
            pltpu.VMEM((BS, Dr), BF16),
            pltpu.VMEM((BS, D), BF16),
            pltpu.VMEM((BS, D), BF16),
            pltpu.VMEM((BS, D), BF16),
            pltpu.VMEM((2, D, NT), jnp.float32),
            pltpu.SemaphoreType.DMA((3,)),
            pltpu.SemaphoreType.DMA((3,)),
            pltpu.SemaphoreType.DMA((2,)),
        ],
        compiler_params=pltpu.CompilerParams(
            collective_id=0,
            vmem_limit_bytes=60 * 1024 * 1024,
        ),
    )(small[0], small[1], small[2], small[3], Wq, small[4], small[5], Wo)


# device time: 90523 ns/iter; 1.3015x vs baseline; 1.0723x over previous
import jax
import jax.numpy as jnp
from jax import lax
from jax.experimental import pallas as pl
from jax.experimental.pallas import tpu as pltpu

B, S, H, Dh, Dr = 2, 512, 16, 128, 32
D = 2048
DC = 128
BS = B * S
SCALE = (Dh + Dr) ** -0.5
BF16 = jnp.bfloat16
NT = 256
NSTRIP = D // NT
KVT = 512


def _mm(a, b):
    return lax.dot_general(a, b, (((1,), (0,)), ((), ())),
                           preferred_element_type=jnp.float32)


def _mmT(a, b):
    return lax.dot_general(a, b, (((1,), (1,)), ((), ())),
                           preferred_element_type=jnp.float32)


def kernel(x, Wdkv, Wuk, Wuv, Wq, Wqr, Wkr, Wo):
    def body(x_ref, wdkv_ref, wuk_ref, wuv_ref, wq_hbm, wqr_ref, wkr_ref,
             wo_hbm, out_ref,
             xb_ref, c_self, c_other, w_send, w_other,
             q_ref, qr_ref, kr_ref, k_ref, v_ref, attn_ref, strip_buf,
             send_sems, recv_sems, strip_sems):
        my_x = lax.axis_index("x")
        my_y = lax.axis_index("y")
        my_z = lax.axis_index("z")
        partner = (1 - my_x, my_y, my_z)

        def strip_cp(hbm_ref, n):
            return pltpu.make_async_copy(
                hbm_ref.at[:, pl.ds(n * NT, NT)],
                strip_buf.at[n % 2], strip_sems.at[n % 2])

        strip_cp(wq_hbm, 0).start()

        barrier = pltpu.get_barrier_semaphore()
        pl.semaphore_signal(barrier, inc=1, device_id=partner,
                            device_id_type=pl.DeviceIdType.MESH)
        pl.semaphore_wait(barrier, 1)

        w_send[0] = wuk_ref[:].astype(BF16)
        w_send[1] = wuv_ref[:].astype(BF16)
        w_rdma = pltpu.make_async_remote_copy(
            src_ref=w_send, dst_ref=w_other,
            send_sem=send_sems.at[0], recv_sem=recv_sems.at[0],
            device_id=partner, device_id_type=pl.DeviceIdType.MESH)
        w_rdma.start()

        wdkv = wdkv_ref[:].astype(BF16)
        for b in range(B):
            xb_ref[pl.ds(b * S, S), :] = x_ref[b].astype(BF16)
            c_self[pl.ds(b * S, S), :] = _mm(
                xb_ref[pl.ds(b * S, S), :], wdkv).astype(BF16)
        c_rdma = pltpu.make_async_remote_copy(
            src_ref=c_self, dst_ref=c_other,
            send_sem=send_sems.at[1], recv_sem=recv_sems.at[1],
            device_id=partner, device_id_type=pl.DeviceIdType.MESH)
        c_rdma.start()

        for n in range(NSTRIP):
            if n + 1 < NSTRIP:
                strip_cp(wq_hbm, n + 1).start()
            strip_cp(wq_hbm, n).wait()
            wqs = strip_buf[n % 2].astype(BF16)
            for b in range(B):
                q_ref[pl.ds(b * S, S), pl.ds(n * NT, NT)] = _mm(
                    xb_ref[pl.ds(b * S, S), :], wqs).astype(BF16)

        wqr = wqr_ref[:].astype(BF16)
        wkr = wkr_ref[:].astype(BF16)
        for b in range(B):
            xb = xb_ref[pl.ds(b * S, S), :]
            qrb = _mm(xb, wqr).astype(BF16)
            for h in range(H):
                qr_ref[h, pl.ds(b * S, S), :] = qrb[:, h * Dr:(h + 1) * Dr]
            kr_ref[pl.ds(b * S, S), :] = _mm(xb, wkr).astype(BF16)

        w_rdma.wait()
        c_rdma.wait()

        for b in range(B):
            cs = c_self[pl.ds(b * S, S), :]
            co = c_other[pl.ds(b * S, S), :]
            for n0 in range(0, D, KVT):
                cols = pl.ds(n0, KVT)
                k_ref[pl.ds(b * S, S), cols] = (
                    _mm(cs, w_send[0, :, cols])
                    + _mm(co, w_other[0, :, cols])).astype(BF16)
                v_ref[pl.ds(b * S, S), cols] = (
                    _mm(cs, w_send[1, :, cols])
                    + _mm(co, w_other[1, :, cols])).astype(BF16)

        strip_cp(wo_hbm, 0).start()

        def attn_step(i, carry):
            b = i // H
            h = i - b * H
            ro = b * S
            co = h * Dh
            q = q_ref[pl.ds(ro, S), pl.ds(co, Dh)]
            k = k_ref[pl.ds(ro, S), pl.ds(co, Dh)]
            qr = qr_ref[h, pl.ds(ro, S), :]
            kr = kr_ref[pl.ds(ro, S), :]
            s = (_mmT(q, k) + _mmT(qr, kr)) * SCALE
            m = jnp.max(s, axis=1, keepdims=True)
            p = jnp.exp(s - m)
            p = p / jnp.sum(p, axis=1, keepdims=True)
            o = _mm(p.astype(BF16), v_ref[pl.ds(ro, S), pl.ds(co, Dh)])
            attn_ref[pl.ds(ro, S), pl.ds(co, Dh)] = o.astype(BF16)
            return carry

        lax.fori_loop(0, B * H, attn_step, 0)

        for n in range(NSTRIP):
            if n + 1 < NSTRIP:
                strip_cp(wo_hbm, n + 1).start()
            strip_cp(wo_hbm, n).wait()
            wos = strip_buf[n % 2].astype(BF16)
            for b in range(B):
                out_ref[b, :, pl.ds(n * NT, NT)] = _mm(
                    attn_ref[pl.ds(b * S, S), :], wos)

    vmem = pl.BlockSpec(memory_space=pltpu.VMEM)
    hbm = pl.BlockSpec(memory_space=pl.ANY)
    return pl.pallas_call(
        body,
        out_shape=jax.ShapeDtypeStruct((B, S, D), jnp.float32),
        in_specs=[vmem, vmem, vmem, vmem, hbm, vmem, vmem, hbm],
        out_specs=vmem,
        scratch_shapes=[
            pltpu.VMEM((BS, D), BF16),
            pltpu.VMEM((BS, DC), BF16),
            pltpu.VMEM((BS, DC), BF16),
            pltpu.VMEM((2, DC, D), BF16),
            pltpu.VMEM((2, DC, D), BF16),
            pltpu.VMEM((BS, D), BF16),
            pltpu.VMEM((H, BS, Dr), BF16),
            pltpu.VMEM((BS, Dr), BF16),
            pltpu.VMEM((BS, D), BF16),
            pltpu.VMEM((BS, D), BF16),
            pltpu.VMEM((BS, D), BF16),
            pltpu.VMEM((2, D, NT), jnp.float32),
            pltpu.SemaphoreType.DMA((2,)),
            pltpu.SemaphoreType.DMA((2,)),
            pltpu.SemaphoreType.DMA((2,)),
        ],
        compiler_params=pltpu.CompilerParams(
            collective_id=0,
            vmem_limit_bytes=61 * 1024 * 1024,
        ),
    )(x, Wdkv, Wuk, Wuv, Wq, Wqr, Wkr, Wo)


# device time: 85406 ns/iter; 1.3795x vs baseline; 1.0599x over previous
import jax
import jax.numpy as jnp
from jax import lax
from jax.experimental import pallas as pl
from jax.experimental.pallas import tpu as pltpu

B, S, H, Dh, Dr = 2, 512, 16, 128, 32
D = 2048
DC = 128
BS = B * S
SCALE = (Dh + Dr) ** -0.5
BF16 = jnp.bfloat16
NT = 256
NSTRIP = D // NT
KVT = 512


def _mm(a, b):
    return lax.dot_general(a, b, (((1,), (0,)), ((), ())),
                           preferred_element_type=jnp.float32)


def _mmT(a, b):
    return lax.dot_general(a, b, (((1,), (1,)), ((), ())),
                           preferred_element_type=jnp.float32)


def kernel(x, Wdkv, Wuk, Wuv, Wq, Wqr, Wkr, Wo):
    def body(x_ref, wdkv_ref, wuk_ref, wuv_ref, wq_hbm, wqr_ref, wkr_ref,
             wo_hbm, out_ref,
             xb_ref, c_self, c_other, w_send, w_other,
             q_ref, qr_ref, kr_ref, k_ref, v_ref, strip_buf, wo_buf,
             send_sems, recv_sems, strip_sems, wo_sems):
        my_x = lax.axis_index("x")
        my_y = lax.axis_index("y")
        my_z = lax.axis_index("z")
        partner = (1 - my_x, my_y, my_z)

        def strip_cp(hbm_ref, n):
            return pltpu.make_async_copy(
                hbm_ref.at[:, pl.ds(n * NT, NT)],
                strip_buf.at[n % 2], strip_sems.at[n % 2])

        strip_cp(wq_hbm, 0).start()

        barrier = pltpu.get_barrier_semaphore()
        pl.semaphore_signal(barrier, inc=1, device_id=partner,
                            device_id_type=pl.DeviceIdType.MESH)
        pl.semaphore_wait(barrier, 1)

        w_send[0] = wuk_ref[:].astype(BF16)
        w_send[1] = wuv_ref[:].astype(BF16)
        w_rdma = pltpu.make_async_remote_copy(
            src_ref=w_send, dst_ref=w_other,
            send_sem=send_sems.at[0], recv_sem=recv_sems.at[0],
            device_id=partner, device_id_type=pl.DeviceIdType.MESH)
        w_rdma.start()

        wdkv = wdkv_ref[:].astype(BF16)
        for b in range(B):
            xb_ref[pl.ds(b * S, S), :] = x_ref[b].astype(BF16)
            c_self[pl.ds(b * S, S), :] = _mm(
                xb_ref[pl.ds(b * S, S), :], wdkv).astype(BF16)
        c_rdma = pltpu.make_async_remote_copy(
            src_ref=c_self, dst_ref=c_other,
            send_sem=send_sems.at[1], recv_sem=recv_sems.at[1],
            device_id=partner, device_id_type=pl.DeviceIdType.MESH)
        c_rdma.start()

        for n in range(NSTRIP):
            if n + 1 < NSTRIP:
                strip_cp(wq_hbm, n + 1).start()
            strip_cp(wq_hbm, n).wait()
            wqs = strip_buf[n % 2].astype(BF16)
            for b in range(B):
                q_ref[pl.ds(b * S, S), pl.ds(n * NT, NT)] = _mm(
                    xb_ref[pl.ds(b * S, S), :], wqs).astype(BF16)

        wqr = wqr_ref[:].astype(BF16)
        wkr = wkr_ref[:].astype(BF16)
        for b in range(B):
            xb = xb_ref[pl.ds(b * S, S), :]
            qrb = _mm(xb, wqr).astype(BF16)
            for h in range(H):
                qr_ref[h, pl.ds(b * S, S), :] = qrb[:, h * Dr:(h + 1) * Dr]
            kr_ref[pl.ds(b * S, S), :] = _mm(xb, wkr).astype(BF16)

        w_rdma.wait()
        c_rdma.wait()

        for b in range(B):
            cs = c_self[pl.ds(b * S, S), :]
            co = c_other[pl.ds(b * S, S), :]
            for n0 in range(0, D, KVT):
                cols = pl.ds(n0, KVT)
                k_ref[pl.ds(b * S, S), cols] = (
                    _mm(cs, w_send[0, :, cols])
                    + _mm(co, w_other[0, :, cols])).astype(BF16)
                v_ref[pl.ds(b * S, S), cols] = (
                    _mm(cs, w_send[1, :, cols])
                    + _mm(co, w_other[1, :, cols])).astype(BF16)

        def wo_cp(h):
            return pltpu.make_async_copy(
                wo_hbm.at[pl.ds(h * Dh, Dh), :],
                wo_buf.at[h % 2], wo_sems.at[h % 2])

        wo_cp(0).start()
        for h in range(H):
            if h + 1 < H:
                wo_cp(h + 1).start()
            wo_cp(h).wait()
            wo_h = wo_buf[h % 2].astype(BF16)
            co = h * Dh
            for b in range(B):
                ro = b * S
                q = q_ref[pl.ds(ro, S), pl.ds(co, Dh)]
                k = k_ref[pl.ds(ro, S), pl.ds(co, Dh)]
                qr = qr_ref[h, pl.ds(ro, S), :]
                kr = kr_ref[pl.ds(ro, S), :]
                p = jnp.exp((_mmT(q, k) + _mmT(qr, kr)) * SCALE)
                r = 1.0 / jnp.sum(p, axis=1, keepdims=True)
                o = _mm(p.astype(BF16), v_ref[pl.ds(ro, S), pl.ds(co, Dh)])
                ob = (o * r).astype(BF16)
                for n0 in range(0, D, D // 2):
                    cols = pl.ds(n0, D // 2)
                    contrib = _mm(ob, wo_h[:, n0:n0 + D // 2])
                    if h == 0:
                        out_ref[b, :, cols] = contrib
                    else:
                        out_ref[b, :, cols] = out_ref[b, :, cols] + contrib

    vmem = pl.BlockSpec(memory_space=pltpu.VMEM)
    hbm = pl.BlockSpec(memory_space=pl.ANY)
    return pl.pallas_call(
        body,
        out_shape=jax.ShapeDtypeStruct((B, S, D), jnp.float32),
        in_specs=[vmem, vmem, vmem, vmem, hbm, vmem, vmem, hbm],
        out_specs=vmem,
        scratch_shapes=[
            pltpu.VMEM((BS, D), BF16),
            pltpu.VMEM((BS, DC), BF16),
            pltpu.VMEM((BS, DC), BF16),
            pltpu.VMEM((2, DC, D), BF16),
            pltpu.VMEM((2, DC, D), BF16),
            pltpu.VMEM((BS, D), BF16),
            pltpu.VMEM((H, BS, Dr), BF16),
            pltpu.VMEM((BS, Dr), BF16),
            pltpu.VMEM((BS, D), BF16),
            pltpu.VMEM((BS, D), BF16),
            pltpu.VMEM((2, D, NT), jnp.float32),
            pltpu.VMEM((2, Dh, D), jnp.float32),
            pltpu.SemaphoreType.DMA((2,)),
            pltpu.SemaphoreType.DMA((2,)),
            pltpu.SemaphoreType.DMA((2,)),
            pltpu.SemaphoreType.DMA((2,)),
        ],
        compiler_params=pltpu.CompilerParams(
            collective_id=0,
            vmem_limit_bytes=61 * 1024 * 1024,
        ),
    )(x, Wdkv, Wuk, Wuv, Wq, Wqr, Wkr, Wo)
